# Initial kernel scaffold; baseline (speedup 1.0000x reference)
import jax
import jax.numpy as jnp
from jax.experimental import pallas as pl


def kernel(x, edge_index, W1, b1, W2, b2, W3, b3, Wc, bc):
    raise NotImplementedError("write your pallas kernel here")


# speedup vs baseline: 6.3663x; 6.3663x over previous
import functools

import jax
import jax.numpy as jnp
from jax import lax
from jax.experimental import pallas as pl
from jax.experimental.pallas import tpu as pltpu
from jax.experimental.pallas import tpu_sc as plsc

N = 10000
NP = 10240
NACC = 10496
NDEG = 10256
TRASH = 10240
E = 320000
EP = 327680
CH = 128
NCHUNK = EP // CH
W = 64
BLK = 256
NBLK = NP // BLK

_f32 = jnp.float32
_i32 = jnp.int32


def _mesh():
    return plsc.VectorSubcoreMesh(core_axis_name="c", subcore_axis_name="s")


_SC_PARAMS = pltpu.CompilerParams(needs_layout_passes=False,
                                  use_tc_tiling_on_sc=False)


def _deg_body(col_hbm, out_hbm, idx_col, acc):
    c = lax.axis_index("c")
    s = lax.axis_index("s")
    w = c * 16 + s
    nck = NCHUNK // 32
    pltpu.sync_copy(col_hbm.at[pl.ds(w * nck, nck)], idx_col)

    @pl.loop(0, NDEG // 16)
    def _(i):
        acc[pl.ds(i * 16, 16)] = jnp.zeros((16,), _f32)

    ones = jnp.ones((16,), _f32)

    @pl.loop(0, nck)
    def _(j):
        for k in range(CH // 16):
            idx = idx_col[j, pl.ds(k * 16, 16)]
            plsc.addupdate_scatter(acc, [idx], ones)

    pltpu.sync_copy(acc.at[pl.ds(0, NP)], out_hbm.at[pl.ds(w * NP, NP)])


def _make_deg():
    return pl.kernel(
        _deg_body,
        out_type=jax.ShapeDtypeStruct((32 * NP,), _f32),
        mesh=_mesh(),
        compiler_params=_SC_PARAMS,
        scratch_types=[
            pltpu.VMEM((NCHUNK // 32, CH), _i32),
            pltpu.VMEM((NDEG,), _f32),
        ],
    )


def _seg_body(passes, split32, hs_hbm, row_hbm, col_hbm, out_hbm,
              idx_row, idx_col, idx_g, bufa, bufb, zbuf, acc, sema, semb):
    c = lax.axis_index("c")
    s = lax.axis_index("s")
    if split32:
        nck = NCHUNK // 32
        base = c * (NCHUNK // 2) + s * nck
    else:
        nck = NCHUNK // 16
        base = s * nck
    pltpu.sync_copy(row_hbm.at[pl.ds(base, nck)], idx_row)
    pltpu.sync_copy(col_hbm.at[pl.ds(base, nck)], idx_col)

    @pl.loop(0, CH)
    def _(i):
        for k in range(W // 16):
            zbuf[i, pl.ds(k * 16, 16)] = jnp.zeros((16,), _f32)

    rows_t = NACC // 16

    for p in range(passes):
        if split32:
            chunk = jnp.int32(0)
            slot = c
        else:
            chunk = c + 2 * p
            slot = chunk
        for k in range(rows_t // CH):
            pltpu.sync_copy(zbuf, acc.at[pl.ds(s * rows_t + k * CH, CH)])
        pltpu.sync_copy(zbuf.at[pl.ds(0, rows_t % CH)],
                        acc.at[pl.ds(s * rows_t + (rows_t // CH) * CH,
                                     rows_t % CH)])
        offv = jnp.full((16,), chunk * NP, _i32)

        @pl.loop(0, nck)
        def _(j):
            for k in range(CH // 16):
                idx_g[j, pl.ds(k * 16, 16)] = (
                    idx_row[j, pl.ds(k * 16, 16)] + offv)

        plsc.subcore_barrier()

        @pl.loop(0, nck, step=2)
        def _(j):
            da = pltpu.async_copy(hs_hbm.at[idx_g.at[j]], bufa, sema)
            db = pltpu.async_copy(hs_hbm.at[idx_g.at[j + 1]], bufb, semb)
            da.wait()
            pltpu.sync_copy(bufa, acc.at[idx_col.at[j]], add=True)
            db.wait()
            pltpu.sync_copy(bufb, acc.at[idx_col.at[j + 1]], add=True)

        plsc.subcore_barrier()
        rows = NP // 16
        pltpu.sync_copy(acc.at[pl.ds(s * rows, rows)],
                        out_hbm.at[pl.ds(slot * NP + s * rows, rows)])
        if p + 1 < passes:
            plsc.subcore_barrier()


def _make_seg(n_slots, passes, split32):
    body = functools.partial(_seg_body, passes, split32)
    nck = NCHUNK // (32 if split32 else 16)
    return pl.kernel(
        body,
        out_type=jax.ShapeDtypeStruct((n_slots * NP, W), _f32),
        mesh=_mesh(),
        compiler_params=_SC_PARAMS,
        scratch_types=[
            pltpu.VMEM((nck, CH), _i32),
            pltpu.VMEM((nck, CH), _i32),
            pltpu.VMEM((nck, CH), _i32),
            pltpu.VMEM((CH, W), _f32),
            pltpu.VMEM((CH, W), _f32),
            pltpu.VMEM((CH, W), _f32),
            pltpu.VMEM_SHARED((NACC, W), _f32),
            pltpu.SemaphoreType.DMA,
            pltpu.SemaphoreType.DMA,
        ],
    )


def _dinv(degp_ref):
    deg = 1.0 + jnp.sum(degp_ref[...], axis=0).reshape(BLK, 1)
    return lax.rsqrt(deg)


def _k1_body(x_ref, w_ref, degp_ref, out_ref):
    dinv = _dinv(degp_ref)
    h = jnp.dot(x_ref[...], w_ref[...], preferred_element_type=_f32)
    for ci in range(8):
        out_ref[ci] = dinv * h[:, ci * W:(ci + 1) * W]


def _mid_body(cin, cout, hs_ref, acc_ref, degp_ref, b_ref, w_ref, out_ref):
    dinv = _dinv(degp_ref)
    accum = jnp.zeros((BLK, cout * W), _f32)
    for ci in range(cin):
        a = jnp.maximum(dinv * (acc_ref[ci] + hs_ref[ci]) + b_ref[ci], 0.0)
        accum = accum + jnp.dot(a, w_ref[ci], preferred_element_type=_f32)
    for co in range(cout):
        out_ref[co] = dinv * accum[:, co * W:(co + 1) * W]


def _k4_body(hs_ref, acc_ref, degp_ref, b_ref, w_ref, bc_ref, out_ref):
    dinv = _dinv(degp_ref)
    acc = acc_ref[0] + acc_ref[1]
    a = jnp.maximum(dinv * (acc + hs_ref[0]) + b_ref[0], 0.0)
    out_ref[...] = (jnp.dot(a, w_ref[...], preferred_element_type=_f32)
                    + bc_ref[0])


def _row_spec(c):
    return pl.BlockSpec((c, BLK, W), lambda i: (0, i, 0))


_DEG_SPEC = pl.BlockSpec((32, BLK), lambda i: (0, i))


def _whole(shape):
    nd = len(shape)
    return pl.BlockSpec(shape, lambda i, _n=nd: (0,) * _n)


def _tc_call(body, in_specs, out_specs, out_shape):
    return pl.pallas_call(
        body, grid=(NBLK,), in_specs=in_specs, out_specs=out_specs,
        out_shape=out_shape)


def kernel(x, edge_index, W1, b1, W2, b2, W3, b3, Wc, bc):
    xp = jnp.zeros((NP, 128), _f32).at[:N].set(x)
    pad = EP - E
    row = jnp.concatenate([edge_index[0], jnp.zeros((pad,), _i32)])
    col = jnp.concatenate([edge_index[1], jnp.full((pad,), TRASH, _i32)])
    row2d = row.reshape(NCHUNK, CH)
    col2d = col.reshape(NCHUNK, CH)

    W2r = W2.reshape(8, W, 256)
    W3r = W3.reshape(4, W, 64)
    Wcp = jnp.zeros((W, 128), _f32).at[:, :3].set(Wc)
    b1r = b1.reshape(8, W)
    b2r = b2.reshape(4, W)
    b3r = b3.reshape(1, W)
    bcp = jnp.zeros((1, 128), _f32).at[0, :3].set(bc)

    degp = _make_deg()(col2d).reshape(32, NP)

    hs1 = _tc_call(
        _k1_body,
        [pl.BlockSpec((BLK, 128), lambda i: (i, 0)), _whole((128, 512)),
         _DEG_SPEC],
        _row_spec(8),
        jax.ShapeDtypeStruct((8, NP, W), _f32))(xp, W1, degp)
    p1 = _make_seg(8, 4, False)(
        hs1.reshape(8 * NP, W), row2d, col2d).reshape(8, NP, W)

    hs2 = _tc_call(
        functools.partial(_mid_body, 8, 4),
        [_row_spec(8), _row_spec(8), _DEG_SPEC, _whole((8, W)),
         _whole((8, W, 256))],
        _row_spec(4),
        jax.ShapeDtypeStruct((4, NP, W), _f32))(hs1, p1, degp, b1r, W2r)
    p2 = _make_seg(4, 2, False)(
        hs2.reshape(4 * NP, W), row2d, col2d).reshape(4, NP, W)

    hs3 = _tc_call(
        functools.partial(_mid_body, 4, 1),
        [_row_spec(4), _row_spec(4), _DEG_SPEC, _whole((4, W)),
         _whole((4, W, 64))],
        _row_spec(1),
        jax.ShapeDtypeStruct((1, NP, W), _f32))(hs2, p2, degp, b2r, W3r)
    p3 = _make_seg(2, 1, True)(
        hs3.reshape(NP, W), row2d, col2d).reshape(2, NP, W)

    out = _tc_call(
        _k4_body,
        [_row_spec(1), _row_spec(2), _DEG_SPEC, _whole((1, W)),
         _whole((W, 128)), _whole((1, 128))],
        pl.BlockSpec((BLK, 128), lambda i: (i, 0)),
        jax.ShapeDtypeStruct((NP, 128), _f32))(hs3, p3, degp, b3r, Wcp, bcp)
    return out[:N, :3]
